# Initial kernel scaffold; baseline (speedup 1.0000x reference)
import jax
import jax.numpy as jnp
from jax.experimental import pallas as pl


def kernel(x_p, x_d, edge_attr_p, edge_attr_d, edge_index_p, edge_index_d, x_p_batch, x_d_batch, params):
    raise NotImplementedError("write your pallas kernel here")


# speedup vs baseline: 2.0254x; 2.0254x over previous
import functools

import jax
import jax.numpy as jnp
from jax import lax
from jax.experimental import pallas as pl
from jax.experimental.pallas import tpu as pltpu
import jax.experimental.pallas.tpu_sc as plsc

N = 10000
E = 160000
D = 128
B = 64

NC = 2
NS = 16
NP = 10112
EP = 163840
EPT = EP // NS
EB = 64
NBLK = EPT // EB
STRIPE = NP // NS
AW = 128


def _edge_mlp_body(ea_ref, w1a_ref, b1a_ref, w1b_ref, b1b_ref, ha_ref, hb_ref):
    ea = ea_ref[...]
    ha_ref[...] = jnp.maximum(ea @ w1a_ref[...] + b1a_ref[...], 0.0)
    hb_ref[...] = jnp.maximum(ea @ w1b_ref[...] + b1b_ref[...], 0.0)


def _edge_mlp(ea_pad, w1a, b1a, w1b, b1b):
    blk = 2048
    grid = EP // blk
    return pl.pallas_call(
        _edge_mlp_body,
        grid=(grid,),
        in_specs=[
            pl.BlockSpec((blk, 16), lambda i: (i, 0)),
            pl.BlockSpec((16, 16), lambda i: (0, 0)),
            pl.BlockSpec((1, 16), lambda i: (0, 0)),
            pl.BlockSpec((16, 16), lambda i: (0, 0)),
            pl.BlockSpec((1, 16), lambda i: (0, 0)),
        ],
        out_specs=[
            pl.BlockSpec((blk, 16), lambda i: (i, 0)),
            pl.BlockSpec((blk, 16), lambda i: (i, 0)),
        ],
        out_shape=[jax.ShapeDtypeStruct((EP, 16), jnp.float32)] * 2,
    )(ea_pad, w1a, b1a, w1b, b1b)


def _node_body(x_ref, seg_ref, w3_ref, root_ref, bias_ref,
               g1w_ref, g1b_ref, g2w_ref, g2b_ref,
               t_ref, r_ref, att_ref):
    x = x_ref[...]
    xr = jnp.maximum(x, 0.0)
    t_ref[...] = xr @ w3_ref[...]
    r_ref[...] = xr @ root_ref[...] + bias_ref[...]
    oh = (seg_ref[...] == lax.broadcasted_iota(jnp.int32, (NP, B), 1)
          ).astype(jnp.float32)
    g1 = jnp.maximum(x @ g1w_ref[...] + g1b_ref[...], 0.0)
    g = g1 @ g2w_ref[...] + g2b_ref[...]
    gm = jnp.where(oh > 0.0, g, -jnp.inf)
    m = jnp.max(gm, axis=0, keepdims=True)
    m = jnp.where(m > -1e30, m, 0.0)
    mn = lax.dot_general(oh, m, (((1,), (1,)), ((), ())))
    e = jnp.exp(g - mn)
    s = lax.dot_general(oh, e, (((0,), (0,)), ((), ())))
    sn = lax.dot_general(oh, s, (((1,), (0,)), ((), ())))
    a = e / (sn + 1e-16)
    att_ref[...] = lax.dot_general(oh, a * x, (((0,), (0,)), ((), ())))


def _node(x_pad, seg_pad, w3aug, root, bias, pool):
    return pl.pallas_call(
        _node_body,
        out_shape=[
            jax.ShapeDtypeStruct((NP, 384), jnp.float32),
            jax.ShapeDtypeStruct((NP, 16), jnp.float32),
            jax.ShapeDtypeStruct((B, D), jnp.float32),
        ],
    )(x_pad, seg_pad, w3aug, root, bias,
      pool['g1_W'], pool['g1_b'].reshape(1, D),
      pool['g2_W'], pool['g2_b'].reshape(1, 1))


def _mid_body(aggp_ref, aggd_ref, rp_ref, rd_ref, w3p_ref, w3d_ref,
              rootp_ref, biasp_ref, rootd_ref, biasd_ref,
              tp_ref, td_ref, r2p_ref, r2d_ref):
    y1p = jnp.maximum(aggp_ref[...] + rp_ref[...], 0.0)
    y1d = jnp.maximum(aggd_ref[...] + rd_ref[...], 0.0)
    tp_ref[...] = y1p @ w3p_ref[...]
    td_ref[...] = y1d @ w3d_ref[...]
    r2p_ref[...] = y1p @ rootp_ref[...] + biasp_ref[...]
    r2d_ref[...] = y1d @ rootd_ref[...] + biasd_ref[...]


def _mid(aggp, aggd, rp, rd, w3p, w3d, rootp, biasp, rootd, biasd):
    blk = NP // 8
    w = lambda i: (0, 0)
    return pl.pallas_call(
        _mid_body,
        grid=(8,),
        in_specs=[
            pl.BlockSpec((blk, 16), lambda i: (i, 0)),
            pl.BlockSpec((blk, 16), lambda i: (i, 0)),
            pl.BlockSpec((blk, 16), lambda i: (i, 0)),
            pl.BlockSpec((blk, 16), lambda i: (i, 0)),
            pl.BlockSpec((16, 384), w),
            pl.BlockSpec((16, 384), w),
            pl.BlockSpec((16, 16), w),
            pl.BlockSpec((1, 16), w),
            pl.BlockSpec((16, 16), w),
            pl.BlockSpec((1, 16), w),
        ],
        out_specs=[
            pl.BlockSpec((blk, 384), lambda i: (i, 0)),
            pl.BlockSpec((blk, 384), lambda i: (i, 0)),
            pl.BlockSpec((blk, 16), lambda i: (i, 0)),
            pl.BlockSpec((blk, 16), lambda i: (i, 0)),
        ],
        out_shape=[
            jax.ShapeDtypeStruct((NP, 384), jnp.float32),
            jax.ShapeDtypeStruct((NP, 384), jnp.float32),
            jax.ShapeDtypeStruct((NP, 16), jnp.float32),
            jax.ShapeDtypeStruct((NP, 16), jnp.float32),
        ],
    )(aggp, aggd, rp, rd, w3p, w3d, rootp, biasp, rootd, biasd)


def _final_body(aggp_ref, aggd_ref, rp_ref, rd_ref, segp_ref, segd_ref,
                attp_ref, attd_ref, l1w_ref, l1b_ref, l2w_ref, l2b_ref,
                out_ref):
    y2p = aggp_ref[...] + rp_ref[...]
    y2d = aggd_ref[...] + rd_ref[...]
    ones = jnp.ones((NP, 1), jnp.float32)

    def mean_pool(y, seg_ref):
        oh = (seg_ref[...] == lax.broadcasted_iota(jnp.int32, (NP, B), 1)
              ).astype(jnp.float32)
        s = lax.dot_general(oh, y, (((0,), (0,)), ((), ())))
        c = lax.dot_general(oh, ones, (((0,), (0,)), ((), ())))
        return s / jnp.maximum(c, 1.0)

    feat = jnp.concatenate(
        [mean_pool(y2p, segp_ref), mean_pool(y2d, segd_ref),
         attp_ref[...], attd_ref[...]], axis=1)
    o1 = feat @ l1w_ref[...] + l1b_ref[...]
    out_ref[...] = o1 @ l2w_ref[...] + l2b_ref[...]


def _final(aggp, aggd, r2p, r2d, segp, segd, attp, attd, p):
    return pl.pallas_call(
        _final_body,
        out_shape=jax.ShapeDtypeStruct((B, 1), jnp.float32),
    )(aggp, aggd, r2p, r2d, segp, segd, attp, attd,
      p['lin1_W'], p['lin1_b'].reshape(1, 8),
      p['lin2_W'], p['lin2_b'].reshape(1, 1))


def _sc_edge_body(zs_hbm, tp_hbm, td_hbm, hp_hbm, hd_hbm,
                  srcp_hbm, dstp_hbm, srcd_hbm, dstd_hbm,
                  outp_hbm, outd_hbm,
                  agg_sh, src_v, dst_v, h_v, t_v, msg_v, sem):
    c = lax.axis_index("c")
    s = lax.axis_index("s")
    row0 = s * STRIPE

    pltpu.sync_copy(zs_hbm.at[pl.ds(row0, STRIPE)],
                    agg_sh.at[pl.ds(row0, STRIPE)])
    pltpu.sync_copy(zs_hbm.at[pl.ds(0, EB)], msg_v)
    plsc.subcore_barrier()

    def run_branch(t_hbm, h_hbm, src_hbm, dst_hbm):
        base = s * EPT

        def blk(i, _):
            off = base + i * EB
            pltpu.sync_copy(src_hbm.at[pl.ds(off, EB)], src_v)
            pltpu.sync_copy(dst_hbm.at[pl.ds(off, EB)], dst_v)
            pltpu.sync_copy(h_hbm.at[pl.ds(off * 16, EB * 16)], h_v)
            pltpu.async_copy(t_hbm.at[src_v], t_v, sem).wait()

            def edge(e, _):
                hv = h_v[pl.ds(e * 16, 16)]
                acc = t_v[e, pl.ds(256, 16)]
                for k in range(16):
                    acc = acc + hv[k] * t_v[e, pl.ds(k * 16, 16)]
                msg_v[e, pl.ds(0, 16)] = acc
                return ()
            lax.fori_loop(0, EB, edge, ())
            pltpu.sync_copy(msg_v, agg_sh.at[dst_v], add=True)
            return ()
        lax.fori_loop(0, NBLK, blk, ())

    @pl.when(c == 0)
    def _():
        run_branch(tp_hbm, hp_hbm, srcp_hbm, dstp_hbm)

    @pl.when(c == 1)
    def _():
        run_branch(td_hbm, hd_hbm, srcd_hbm, dstd_hbm)

    plsc.subcore_barrier()

    @pl.when(c == 0)
    def _():
        pltpu.sync_copy(agg_sh.at[pl.ds(row0, STRIPE)],
                        outp_hbm.at[pl.ds(row0, STRIPE)])

    @pl.when(c == 1)
    def _():
        pltpu.sync_copy(agg_sh.at[pl.ds(row0, STRIPE)],
                        outd_hbm.at[pl.ds(row0, STRIPE)])


@functools.cache
def _sc_edge():
    return pl.kernel(
        _sc_edge_body,
        out_type=[jax.ShapeDtypeStruct((NP, AW), jnp.float32),
                  jax.ShapeDtypeStruct((NP, AW), jnp.float32)],
        mesh=plsc.VectorSubcoreMesh(core_axis_name="c", subcore_axis_name="s",
                                    num_cores=NC, num_subcores=NS),
        scratch_types=[
            pltpu.VMEM_SHARED((NP, AW), jnp.float32),
            pltpu.VMEM((EB,), jnp.int32),
            pltpu.VMEM((EB,), jnp.int32),
            pltpu.VMEM((EB * 16,), jnp.float32),
            pltpu.VMEM((EB, 384), jnp.float32),
            pltpu.VMEM((EB, AW), jnp.float32),
            pltpu.SemaphoreType.DMA,
        ],
    )


def _w3aug(cp, ic):
    w3 = cp['nW2'].reshape(16, ic, 16).transpose(1, 0, 2).reshape(ic, 256)
    return jnp.concatenate(
        [w3, cp['nb2'].reshape(ic, 16), jnp.zeros((ic, 112), jnp.float32)],
        axis=1)


def kernel(x_p, x_d, edge_attr_p, edge_attr_d, edge_index_p, edge_index_d,
           x_p_batch, x_d_batch, params):
    f32 = jnp.float32
    cv_p = params['convs_d']
    cv_d = params['convs_p']

    xp = jnp.concatenate([x_p, jnp.zeros((NP - N, D), f32)], axis=0)
    xd = jnp.concatenate([x_d, jnp.zeros((NP - N, D), f32)], axis=0)
    segp = jnp.concatenate(
        [x_p_batch, jnp.full((NP - N,), B, jnp.int32)]).reshape(NP, 1)
    segd = jnp.concatenate(
        [x_d_batch, jnp.full((NP - N,), B, jnp.int32)]).reshape(NP, 1)
    eap = jnp.concatenate([edge_attr_p, jnp.zeros((EP - E, 16), f32)], axis=0)
    ead = jnp.concatenate([edge_attr_d, jnp.zeros((EP - E, 16), f32)], axis=0)

    def pad_idx(v, fill):
        return jnp.concatenate([v, jnp.full((EP - E,), fill, jnp.int32)])
    srcp = pad_idx(edge_index_p[0], N)
    dstp = pad_idx(edge_index_p[1], N)
    srcd = pad_idx(edge_index_d[0], N)
    dstd = pad_idx(edge_index_d[1], N)

    h1p, h2p = _edge_mlp(eap, cv_p[0]['nW1'], cv_p[0]['nb1'].reshape(1, 16),
                         cv_p[1]['nW1'], cv_p[1]['nb1'].reshape(1, 16))
    h1d, h2d = _edge_mlp(ead, cv_d[0]['nW1'], cv_d[0]['nb1'].reshape(1, 16),
                         cv_d[1]['nW1'], cv_d[1]['nb1'].reshape(1, 16))

    t1p, r1p, attp = _node(xp, segp, _w3aug(cv_p[0], D), cv_p[0]['root'],
                           cv_p[0]['bias'].reshape(1, 16), params['pool'])
    t1d, r1d, attd = _node(xd, segd, _w3aug(cv_d[0], D), cv_d[0]['root'],
                           cv_d[0]['bias'].reshape(1, 16), params['pool'])

    zs = jnp.zeros((NP, AW), f32)
    agg1p, agg1d = _sc_edge()(zs, t1p, t1d,
                              h1p.reshape(EP * 16), h1d.reshape(EP * 16),
                              srcp, dstp, srcd, dstd)

    t2p, t2d, r2p, r2d = _mid(
        agg1p[:, :16], agg1d[:, :16], r1p, r1d,
        _w3aug(cv_p[1], 16), _w3aug(cv_d[1], 16),
        cv_p[1]['root'], cv_p[1]['bias'].reshape(1, 16),
        cv_d[1]['root'], cv_d[1]['bias'].reshape(1, 16))

    agg2p, agg2d = _sc_edge()(zs, t2p, t2d,
                              h2p.reshape(EP * 16), h2d.reshape(EP * 16),
                              srcp, dstp, srcd, dstd)

    return _final(agg2p[:, :16], agg2d[:, :16], r2p, r2d, segp, segd,
                  attp, attd, params)


# speedup vs baseline: 2.1402x; 1.0567x over previous
import functools

import jax
import jax.numpy as jnp
from jax import lax
from jax.experimental import pallas as pl
from jax.experimental.pallas import tpu as pltpu
import jax.experimental.pallas.tpu_sc as plsc

N = 10000
E = 160000
D = 128
B = 64

NC = 2
NS = 16
NP = 10112
EP = 163840
EPT = EP // NS
EB = 64
NBLK = EPT // EB
STRIPE = NP // NS
AW = 128


def _edge_mlp_body(ea_ref, w1a_ref, b1a_ref, w1b_ref, b1b_ref, ha_ref, hb_ref):
    ea = ea_ref[...]
    ha_ref[...] = jnp.maximum(ea @ w1a_ref[...] + b1a_ref[...], 0.0)
    hb_ref[...] = jnp.maximum(ea @ w1b_ref[...] + b1b_ref[...], 0.0)


def _edge_mlp(ea_pad, w1a, b1a, w1b, b1b):
    blk = 2048
    grid = EP // blk
    return pl.pallas_call(
        _edge_mlp_body,
        grid=(grid,),
        in_specs=[
            pl.BlockSpec((blk, 16), lambda i: (i, 0)),
            pl.BlockSpec((16, 16), lambda i: (0, 0)),
            pl.BlockSpec((1, 16), lambda i: (0, 0)),
            pl.BlockSpec((16, 16), lambda i: (0, 0)),
            pl.BlockSpec((1, 16), lambda i: (0, 0)),
        ],
        out_specs=[
            pl.BlockSpec((blk, 16), lambda i: (i, 0)),
            pl.BlockSpec((blk, 16), lambda i: (i, 0)),
        ],
        out_shape=[jax.ShapeDtypeStruct((EP, 16), jnp.float32)] * 2,
    )(ea_pad, w1a, b1a, w1b, b1b)


def _node_body(x_ref, seg_ref, w3_ref, root_ref, bias_ref,
               g1w_ref, g1b_ref, g2w_ref, g2b_ref,
               t_ref, r_ref, att_ref):
    x = x_ref[...]
    xr = jnp.maximum(x, 0.0)
    t_ref[...] = xr @ w3_ref[...]
    r_ref[...] = xr @ root_ref[...] + bias_ref[...]
    oh = (seg_ref[...] == lax.broadcasted_iota(jnp.int32, (NP, B), 1)
          ).astype(jnp.float32)
    g1 = jnp.maximum(x @ g1w_ref[...] + g1b_ref[...], 0.0)
    g = g1 @ g2w_ref[...] + g2b_ref[...]
    gm = jnp.where(oh > 0.0, g, -jnp.inf)
    m = jnp.max(gm, axis=0, keepdims=True)
    m = jnp.where(m > -1e30, m, 0.0)
    mn = lax.dot_general(oh, m, (((1,), (1,)), ((), ())))
    e = jnp.exp(g - mn)
    s = lax.dot_general(oh, e, (((0,), (0,)), ((), ())))
    sn = lax.dot_general(oh, s, (((1,), (0,)), ((), ())))
    a = e / (sn + 1e-16)
    att_ref[...] = lax.dot_general(oh, a * x, (((0,), (0,)), ((), ())))


def _node(x_pad, seg_pad, w3aug, root, bias, pool):
    return pl.pallas_call(
        _node_body,
        out_shape=[
            jax.ShapeDtypeStruct((NP, 384), jnp.float32),
            jax.ShapeDtypeStruct((NP, 16), jnp.float32),
            jax.ShapeDtypeStruct((B, D), jnp.float32),
        ],
    )(x_pad, seg_pad, w3aug, root, bias,
      pool['g1_W'], pool['g1_b'].reshape(1, D),
      pool['g2_W'], pool['g2_b'].reshape(1, 1))


def _mid_body(aggp_ref, aggd_ref, rp_ref, rd_ref, w3p_ref, w3d_ref,
              rootp_ref, biasp_ref, rootd_ref, biasd_ref,
              tp_ref, td_ref, r2p_ref, r2d_ref):
    y1p = jnp.maximum(aggp_ref[...] + rp_ref[...], 0.0)
    y1d = jnp.maximum(aggd_ref[...] + rd_ref[...], 0.0)
    tp_ref[...] = y1p @ w3p_ref[...]
    td_ref[...] = y1d @ w3d_ref[...]
    r2p_ref[...] = y1p @ rootp_ref[...] + biasp_ref[...]
    r2d_ref[...] = y1d @ rootd_ref[...] + biasd_ref[...]


def _mid(aggp, aggd, rp, rd, w3p, w3d, rootp, biasp, rootd, biasd):
    blk = NP // 8
    w = lambda i: (0, 0)
    return pl.pallas_call(
        _mid_body,
        grid=(8,),
        in_specs=[
            pl.BlockSpec((blk, 16), lambda i: (i, 0)),
            pl.BlockSpec((blk, 16), lambda i: (i, 0)),
            pl.BlockSpec((blk, 16), lambda i: (i, 0)),
            pl.BlockSpec((blk, 16), lambda i: (i, 0)),
            pl.BlockSpec((16, 384), w),
            pl.BlockSpec((16, 384), w),
            pl.BlockSpec((16, 16), w),
            pl.BlockSpec((1, 16), w),
            pl.BlockSpec((16, 16), w),
            pl.BlockSpec((1, 16), w),
        ],
        out_specs=[
            pl.BlockSpec((blk, 384), lambda i: (i, 0)),
            pl.BlockSpec((blk, 384), lambda i: (i, 0)),
            pl.BlockSpec((blk, 16), lambda i: (i, 0)),
            pl.BlockSpec((blk, 16), lambda i: (i, 0)),
        ],
        out_shape=[
            jax.ShapeDtypeStruct((NP, 384), jnp.float32),
            jax.ShapeDtypeStruct((NP, 384), jnp.float32),
            jax.ShapeDtypeStruct((NP, 16), jnp.float32),
            jax.ShapeDtypeStruct((NP, 16), jnp.float32),
        ],
    )(aggp, aggd, rp, rd, w3p, w3d, rootp, biasp, rootd, biasd)


def _final_body(aggp_ref, aggd_ref, rp_ref, rd_ref, segp_ref, segd_ref,
                attp_ref, attd_ref, l1w_ref, l1b_ref, l2w_ref, l2b_ref,
                out_ref):
    y2p = aggp_ref[...] + rp_ref[...]
    y2d = aggd_ref[...] + rd_ref[...]
    ones = jnp.ones((NP, 1), jnp.float32)

    def mean_pool(y, seg_ref):
        oh = (seg_ref[...] == lax.broadcasted_iota(jnp.int32, (NP, B), 1)
              ).astype(jnp.float32)
        s = lax.dot_general(oh, y, (((0,), (0,)), ((), ())))
        c = lax.dot_general(oh, ones, (((0,), (0,)), ((), ())))
        return s / jnp.maximum(c, 1.0)

    feat = jnp.concatenate(
        [mean_pool(y2p, segp_ref), mean_pool(y2d, segd_ref),
         attp_ref[...], attd_ref[...]], axis=1)
    o1 = feat @ l1w_ref[...] + l1b_ref[...]
    out_ref[...] = o1 @ l2w_ref[...] + l2b_ref[...]


def _final(aggp, aggd, r2p, r2d, segp, segd, attp, attd, p):
    return pl.pallas_call(
        _final_body,
        out_shape=jax.ShapeDtypeStruct((B, 1), jnp.float32),
    )(aggp, aggd, r2p, r2d, segp, segd, attp, attd,
      p['lin1_W'], p['lin1_b'].reshape(1, 8),
      p['lin2_W'], p['lin2_b'].reshape(1, 1))


def _sc_edge_body(zs_hbm, tp_hbm, td_hbm, hp_hbm, hd_hbm,
                  srcp_hbm, dstp_hbm, srcd_hbm, dstd_hbm,
                  outp_hbm, outd_hbm,
                  agg_sh, src_v, dst_v, h_s, t_v, msg_v, sem):
    c = lax.axis_index("c")
    s = lax.axis_index("s")
    row0 = s * STRIPE

    pltpu.sync_copy(zs_hbm.at[pl.ds(row0, STRIPE)],
                    agg_sh.at[pl.ds(row0, STRIPE)])
    pltpu.sync_copy(zs_hbm.at[pl.ds(0, EB)], msg_v)
    plsc.subcore_barrier()

    def run_branch(t_hbm, h_hbm, src_hbm, dst_hbm):
        base = s * EPT

        def blk(i, _):
            off = base + i * EB
            pltpu.sync_copy(src_hbm.at[pl.ds(off, EB)], src_v)
            pltpu.sync_copy(dst_hbm.at[pl.ds(off, EB)], dst_v)
            pltpu.sync_copy(h_hbm.at[pl.ds(off * 16, EB * 16)], h_s)
            pltpu.async_copy(t_hbm.at[src_v], t_v, sem).wait()

            def edge(e, _):
                hv = h_s[pl.ds(e * 16, 16)]
                a0 = t_v[e, pl.ds(256, 16)]
                a1 = hv[1] * t_v[e, pl.ds(16, 16)]
                a2 = hv[2] * t_v[e, pl.ds(32, 16)]
                a3 = hv[3] * t_v[e, pl.ds(48, 16)]
                a0 = a0 + hv[0] * t_v[e, pl.ds(0, 16)]
                for k in range(4, 16, 4):
                    a0 = a0 + hv[k] * t_v[e, pl.ds(k * 16, 16)]
                    a1 = a1 + hv[k + 1] * t_v[e, pl.ds((k + 1) * 16, 16)]
                    a2 = a2 + hv[k + 2] * t_v[e, pl.ds((k + 2) * 16, 16)]
                    a3 = a3 + hv[k + 3] * t_v[e, pl.ds((k + 3) * 16, 16)]
                msg_v[e, pl.ds(0, 16)] = (a0 + a1) + (a2 + a3)
                return ()
            lax.fori_loop(0, EB, edge, ())
            pltpu.sync_copy(msg_v, agg_sh.at[dst_v], add=True)
            return ()
        lax.fori_loop(0, NBLK, blk, ())

    @pl.when(c == 0)
    def _():
        run_branch(tp_hbm, hp_hbm, srcp_hbm, dstp_hbm)

    @pl.when(c == 1)
    def _():
        run_branch(td_hbm, hd_hbm, srcd_hbm, dstd_hbm)

    plsc.subcore_barrier()

    @pl.when(c == 0)
    def _():
        pltpu.sync_copy(agg_sh.at[pl.ds(row0, STRIPE)],
                        outp_hbm.at[pl.ds(row0, STRIPE)])

    @pl.when(c == 1)
    def _():
        pltpu.sync_copy(agg_sh.at[pl.ds(row0, STRIPE)],
                        outd_hbm.at[pl.ds(row0, STRIPE)])


@functools.cache
def _sc_edge():
    return pl.kernel(
        _sc_edge_body,
        out_type=[jax.ShapeDtypeStruct((NP, AW), jnp.float32),
                  jax.ShapeDtypeStruct((NP, AW), jnp.float32)],
        mesh=plsc.VectorSubcoreMesh(core_axis_name="c", subcore_axis_name="s",
                                    num_cores=NC, num_subcores=NS),
        scratch_types=[
            pltpu.VMEM_SHARED((NP, AW), jnp.float32),
            pltpu.VMEM((EB,), jnp.int32),
            pltpu.VMEM((EB,), jnp.int32),
            pltpu.VMEM((EB * 16,), jnp.float32),
            pltpu.VMEM((EB, 384), jnp.float32),
            pltpu.VMEM((EB, AW), jnp.float32),
            pltpu.SemaphoreType.DMA,
        ],
    )


def _w3aug(cp, ic):
    w3 = cp['nW2'].reshape(16, ic, 16).transpose(1, 0, 2).reshape(ic, 256)
    return jnp.concatenate(
        [w3, cp['nb2'].reshape(ic, 16), jnp.zeros((ic, 112), jnp.float32)],
        axis=1)


def kernel(x_p, x_d, edge_attr_p, edge_attr_d, edge_index_p, edge_index_d,
           x_p_batch, x_d_batch, params):
    f32 = jnp.float32
    cv_p = params['convs_d']
    cv_d = params['convs_p']

    xp = jnp.concatenate([x_p, jnp.zeros((NP - N, D), f32)], axis=0)
    xd = jnp.concatenate([x_d, jnp.zeros((NP - N, D), f32)], axis=0)
    segp = jnp.concatenate(
        [x_p_batch, jnp.full((NP - N,), B, jnp.int32)]).reshape(NP, 1)
    segd = jnp.concatenate(
        [x_d_batch, jnp.full((NP - N,), B, jnp.int32)]).reshape(NP, 1)
    eap = jnp.concatenate([edge_attr_p, jnp.zeros((EP - E, 16), f32)], axis=0)
    ead = jnp.concatenate([edge_attr_d, jnp.zeros((EP - E, 16), f32)], axis=0)

    def pad_idx(v, fill):
        return jnp.concatenate([v, jnp.full((EP - E,), fill, jnp.int32)])
    srcp = pad_idx(edge_index_p[0], N)
    dstp = pad_idx(edge_index_p[1], N)
    srcd = pad_idx(edge_index_d[0], N)
    dstd = pad_idx(edge_index_d[1], N)

    h1p, h2p = _edge_mlp(eap, cv_p[0]['nW1'], cv_p[0]['nb1'].reshape(1, 16),
                         cv_p[1]['nW1'], cv_p[1]['nb1'].reshape(1, 16))
    h1d, h2d = _edge_mlp(ead, cv_d[0]['nW1'], cv_d[0]['nb1'].reshape(1, 16),
                         cv_d[1]['nW1'], cv_d[1]['nb1'].reshape(1, 16))

    t1p, r1p, attp = _node(xp, segp, _w3aug(cv_p[0], D), cv_p[0]['root'],
                           cv_p[0]['bias'].reshape(1, 16), params['pool'])
    t1d, r1d, attd = _node(xd, segd, _w3aug(cv_d[0], D), cv_d[0]['root'],
                           cv_d[0]['bias'].reshape(1, 16), params['pool'])

    zs = jnp.zeros((NP, AW), f32)
    agg1p, agg1d = _sc_edge()(zs, t1p, t1d,
                              h1p.reshape(EP * 16), h1d.reshape(EP * 16),
                              srcp, dstp, srcd, dstd)

    t2p, t2d, r2p, r2d = _mid(
        agg1p[:, :16], agg1d[:, :16], r1p, r1d,
        _w3aug(cv_p[1], 16), _w3aug(cv_d[1], 16),
        cv_p[1]['root'], cv_p[1]['bias'].reshape(1, 16),
        cv_d[1]['root'], cv_d[1]['bias'].reshape(1, 16))

    agg2p, agg2d = _sc_edge()(zs, t2p, t2d,
                              h2p.reshape(EP * 16), h2d.reshape(EP * 16),
                              srcp, dstp, srcd, dstd)

    return _final(agg2p[:, :16], agg2d[:, :16], r2p, r2d, segp, segd,
                  attp, attd, params)


# speedup vs baseline: 2.1421x; 1.0009x over previous
import functools

import jax
import jax.numpy as jnp
from jax import lax
from jax.experimental import pallas as pl
from jax.experimental.pallas import tpu as pltpu
import jax.experimental.pallas.tpu_sc as plsc

N = 10000
E = 160000
D = 128
B = 64

NC = 2
NS = 16
NP = 10112
EP = 163840
EPT = EP // NS
EB = 64
NBLK = EPT // EB
STRIPE = NP // NS
AW = 128


def _edge_mlp_body(ea_ref, w1a_ref, b1a_ref, w1b_ref, b1b_ref, ha_ref, hb_ref):
    ea = ea_ref[...]
    ha_ref[...] = jnp.maximum(ea @ w1a_ref[...] + b1a_ref[...], 0.0)
    hb_ref[...] = jnp.maximum(ea @ w1b_ref[...] + b1b_ref[...], 0.0)


def _edge_mlp(ea_pad, w1a, b1a, w1b, b1b):
    blk = 2048
    grid = EP // blk
    return pl.pallas_call(
        _edge_mlp_body,
        grid=(grid,),
        in_specs=[
            pl.BlockSpec((blk, 16), lambda i: (i, 0)),
            pl.BlockSpec((16, 16), lambda i: (0, 0)),
            pl.BlockSpec((1, 16), lambda i: (0, 0)),
            pl.BlockSpec((16, 16), lambda i: (0, 0)),
            pl.BlockSpec((1, 16), lambda i: (0, 0)),
        ],
        out_specs=[
            pl.BlockSpec((blk, 16), lambda i: (i, 0)),
            pl.BlockSpec((blk, 16), lambda i: (i, 0)),
        ],
        out_shape=[jax.ShapeDtypeStruct((EP, 16), jnp.float32)] * 2,
    )(ea_pad, w1a, b1a, w1b, b1b)


def _node_body(x_ref, seg_ref, w3_ref, root_ref, bias_ref,
               g1w_ref, g1b_ref, g2w_ref, g2b_ref,
               t_ref, r_ref, att_ref):
    x = x_ref[...]
    xr = jnp.maximum(x, 0.0)
    t_ref[...] = xr @ w3_ref[...]
    r_ref[...] = xr @ root_ref[...] + bias_ref[...]
    oh = (seg_ref[...] == lax.broadcasted_iota(jnp.int32, (NP, B), 1)
          ).astype(jnp.float32)
    g1 = jnp.maximum(x @ g1w_ref[...] + g1b_ref[...], 0.0)
    g = g1 @ g2w_ref[...] + g2b_ref[...]
    gm = jnp.where(oh > 0.0, g, -jnp.inf)
    m = jnp.max(gm, axis=0, keepdims=True)
    m = jnp.where(m > -1e30, m, 0.0)
    mn = lax.dot_general(oh, m, (((1,), (1,)), ((), ())))
    e = jnp.exp(g - mn)
    s = lax.dot_general(oh, e, (((0,), (0,)), ((), ())))
    sn = lax.dot_general(oh, s, (((1,), (0,)), ((), ())))
    a = e / (sn + 1e-16)
    att_ref[...] = lax.dot_general(oh, a * x, (((0,), (0,)), ((), ())))


def _node(x_pad, seg_pad, w3aug, root, bias, pool):
    return pl.pallas_call(
        _node_body,
        out_shape=[
            jax.ShapeDtypeStruct((NP, 384), jnp.float32),
            jax.ShapeDtypeStruct((NP, 16), jnp.float32),
            jax.ShapeDtypeStruct((B, D), jnp.float32),
        ],
    )(x_pad, seg_pad, w3aug, root, bias,
      pool['g1_W'], pool['g1_b'].reshape(1, D),
      pool['g2_W'], pool['g2_b'].reshape(1, 1))


def _mid_body(aggp_ref, aggd_ref, rp_ref, rd_ref, w3p_ref, w3d_ref,
              rootp_ref, biasp_ref, rootd_ref, biasd_ref,
              tp_ref, td_ref, r2p_ref, r2d_ref):
    y1p = jnp.maximum(aggp_ref[...] + rp_ref[...], 0.0)
    y1d = jnp.maximum(aggd_ref[...] + rd_ref[...], 0.0)
    tp_ref[...] = y1p @ w3p_ref[...]
    td_ref[...] = y1d @ w3d_ref[...]
    r2p_ref[...] = y1p @ rootp_ref[...] + biasp_ref[...]
    r2d_ref[...] = y1d @ rootd_ref[...] + biasd_ref[...]


def _mid(aggp, aggd, rp, rd, w3p, w3d, rootp, biasp, rootd, biasd):
    blk = NP // 8
    w = lambda i: (0, 0)
    return pl.pallas_call(
        _mid_body,
        grid=(8,),
        in_specs=[
            pl.BlockSpec((blk, 16), lambda i: (i, 0)),
            pl.BlockSpec((blk, 16), lambda i: (i, 0)),
            pl.BlockSpec((blk, 16), lambda i: (i, 0)),
            pl.BlockSpec((blk, 16), lambda i: (i, 0)),
            pl.BlockSpec((16, 384), w),
            pl.BlockSpec((16, 384), w),
            pl.BlockSpec((16, 16), w),
            pl.BlockSpec((1, 16), w),
            pl.BlockSpec((16, 16), w),
            pl.BlockSpec((1, 16), w),
        ],
        out_specs=[
            pl.BlockSpec((blk, 384), lambda i: (i, 0)),
            pl.BlockSpec((blk, 384), lambda i: (i, 0)),
            pl.BlockSpec((blk, 16), lambda i: (i, 0)),
            pl.BlockSpec((blk, 16), lambda i: (i, 0)),
        ],
        out_shape=[
            jax.ShapeDtypeStruct((NP, 384), jnp.float32),
            jax.ShapeDtypeStruct((NP, 384), jnp.float32),
            jax.ShapeDtypeStruct((NP, 16), jnp.float32),
            jax.ShapeDtypeStruct((NP, 16), jnp.float32),
        ],
    )(aggp, aggd, rp, rd, w3p, w3d, rootp, biasp, rootd, biasd)


def _final_body(aggp_ref, aggd_ref, rp_ref, rd_ref, segp_ref, segd_ref,
                attp_ref, attd_ref, l1w_ref, l1b_ref, l2w_ref, l2b_ref,
                out_ref):
    y2p = aggp_ref[...] + rp_ref[...]
    y2d = aggd_ref[...] + rd_ref[...]
    ones = jnp.ones((NP, 1), jnp.float32)

    def mean_pool(y, seg_ref):
        oh = (seg_ref[...] == lax.broadcasted_iota(jnp.int32, (NP, B), 1)
              ).astype(jnp.float32)
        s = lax.dot_general(oh, y, (((0,), (0,)), ((), ())))
        c = lax.dot_general(oh, ones, (((0,), (0,)), ((), ())))
        return s / jnp.maximum(c, 1.0)

    feat = jnp.concatenate(
        [mean_pool(y2p, segp_ref), mean_pool(y2d, segd_ref),
         attp_ref[...], attd_ref[...]], axis=1)
    o1 = feat @ l1w_ref[...] + l1b_ref[...]
    out_ref[...] = o1 @ l2w_ref[...] + l2b_ref[...]


def _final(aggp, aggd, r2p, r2d, segp, segd, attp, attd, p):
    return pl.pallas_call(
        _final_body,
        out_shape=jax.ShapeDtypeStruct((B, 1), jnp.float32),
    )(aggp, aggd, r2p, r2d, segp, segd, attp, attd,
      p['lin1_W'], p['lin1_b'].reshape(1, 8),
      p['lin2_W'], p['lin2_b'].reshape(1, 1))


def _sc_edge_body(zs_hbm, tp_hbm, td_hbm, hp_hbm, hd_hbm,
                  srcp_hbm, dstp_hbm, srcd_hbm, dstd_hbm,
                  outp_hbm, outd_hbm,
                  agg_sh, src_v, dst_v, h_s, t_v, msg_v, sem):
    c = lax.axis_index("c")
    s = lax.axis_index("s")
    row0 = s * STRIPE

    pltpu.sync_copy(zs_hbm.at[pl.ds(row0, STRIPE)],
                    agg_sh.at[pl.ds(row0, STRIPE)])
    pltpu.sync_copy(zs_hbm.at[pl.ds(0, EB)], msg_v)
    plsc.subcore_barrier()

    def run_branch(t_hbm, h_hbm, src_hbm, dst_hbm):
        base = s * EPT

        def blk(i, _):
            off = base + i * EB
            pltpu.sync_copy(src_hbm.at[pl.ds(off, EB)], src_v)
            pltpu.sync_copy(dst_hbm.at[pl.ds(off, EB)], dst_v)
            pltpu.sync_copy(h_hbm.at[pl.ds(off * 16, EB * 16)], h_s)
            pltpu.async_copy(t_hbm.at[src_v], t_v, sem).wait()

            def edge4(q, _):
                for u in range(4):
                    e = q * 4 + u
                    hv = h_s[pl.ds(e * 16, 16)]
                    a0 = t_v[e, pl.ds(256, 16)]
                    a1 = hv[1] * t_v[e, pl.ds(16, 16)]
                    a0 = a0 + hv[0] * t_v[e, pl.ds(0, 16)]
                    for k in range(2, 16, 2):
                        a0 = a0 + hv[k] * t_v[e, pl.ds(k * 16, 16)]
                        a1 = a1 + hv[k + 1] * t_v[e, pl.ds((k + 1) * 16, 16)]
                    msg_v[e, pl.ds(0, 16)] = a0 + a1
                return ()
            lax.fori_loop(0, EB // 4, edge4, ())
            pltpu.sync_copy(msg_v, agg_sh.at[dst_v], add=True)
            return ()
        lax.fori_loop(0, NBLK, blk, ())

    @pl.when(c == 0)
    def _():
        run_branch(tp_hbm, hp_hbm, srcp_hbm, dstp_hbm)

    @pl.when(c == 1)
    def _():
        run_branch(td_hbm, hd_hbm, srcd_hbm, dstd_hbm)

    plsc.subcore_barrier()

    @pl.when(c == 0)
    def _():
        pltpu.sync_copy(agg_sh.at[pl.ds(row0, STRIPE)],
                        outp_hbm.at[pl.ds(row0, STRIPE)])

    @pl.when(c == 1)
    def _():
        pltpu.sync_copy(agg_sh.at[pl.ds(row0, STRIPE)],
                        outd_hbm.at[pl.ds(row0, STRIPE)])


@functools.cache
def _sc_edge():
    return pl.kernel(
        _sc_edge_body,
        out_type=[jax.ShapeDtypeStruct((NP, AW), jnp.float32),
                  jax.ShapeDtypeStruct((NP, AW), jnp.float32)],
        mesh=plsc.VectorSubcoreMesh(core_axis_name="c", subcore_axis_name="s",
                                    num_cores=NC, num_subcores=NS),
        scratch_types=[
            pltpu.VMEM_SHARED((NP, AW), jnp.float32),
            pltpu.VMEM((EB,), jnp.int32),
            pltpu.VMEM((EB,), jnp.int32),
            pltpu.VMEM((EB * 16,), jnp.float32),
            pltpu.VMEM((EB, 384), jnp.float32),
            pltpu.VMEM((EB, AW), jnp.float32),
            pltpu.SemaphoreType.DMA,
        ],
    )


def _w3aug(cp, ic):
    w3 = cp['nW2'].reshape(16, ic, 16).transpose(1, 0, 2).reshape(ic, 256)
    return jnp.concatenate(
        [w3, cp['nb2'].reshape(ic, 16), jnp.zeros((ic, 112), jnp.float32)],
        axis=1)


def kernel(x_p, x_d, edge_attr_p, edge_attr_d, edge_index_p, edge_index_d,
           x_p_batch, x_d_batch, params):
    f32 = jnp.float32
    cv_p = params['convs_d']
    cv_d = params['convs_p']

    xp = jnp.concatenate([x_p, jnp.zeros((NP - N, D), f32)], axis=0)
    xd = jnp.concatenate([x_d, jnp.zeros((NP - N, D), f32)], axis=0)
    segp = jnp.concatenate(
        [x_p_batch, jnp.full((NP - N,), B, jnp.int32)]).reshape(NP, 1)
    segd = jnp.concatenate(
        [x_d_batch, jnp.full((NP - N,), B, jnp.int32)]).reshape(NP, 1)
    eap = jnp.concatenate([edge_attr_p, jnp.zeros((EP - E, 16), f32)], axis=0)
    ead = jnp.concatenate([edge_attr_d, jnp.zeros((EP - E, 16), f32)], axis=0)

    def pad_idx(v, fill):
        return jnp.concatenate([v, jnp.full((EP - E,), fill, jnp.int32)])
    srcp = pad_idx(edge_index_p[0], N)
    dstp = pad_idx(edge_index_p[1], N)
    srcd = pad_idx(edge_index_d[0], N)
    dstd = pad_idx(edge_index_d[1], N)

    h1p, h2p = _edge_mlp(eap, cv_p[0]['nW1'], cv_p[0]['nb1'].reshape(1, 16),
                         cv_p[1]['nW1'], cv_p[1]['nb1'].reshape(1, 16))
    h1d, h2d = _edge_mlp(ead, cv_d[0]['nW1'], cv_d[0]['nb1'].reshape(1, 16),
                         cv_d[1]['nW1'], cv_d[1]['nb1'].reshape(1, 16))

    t1p, r1p, attp = _node(xp, segp, _w3aug(cv_p[0], D), cv_p[0]['root'],
                           cv_p[0]['bias'].reshape(1, 16), params['pool'])
    t1d, r1d, attd = _node(xd, segd, _w3aug(cv_d[0], D), cv_d[0]['root'],
                           cv_d[0]['bias'].reshape(1, 16), params['pool'])

    zs = jnp.zeros((NP, AW), f32)
    agg1p, agg1d = _sc_edge()(zs, t1p, t1d,
                              h1p.reshape(EP * 16), h1d.reshape(EP * 16),
                              srcp, dstp, srcd, dstd)

    t2p, t2d, r2p, r2d = _mid(
        agg1p[:, :16], agg1d[:, :16], r1p, r1d,
        _w3aug(cv_p[1], 16), _w3aug(cv_d[1], 16),
        cv_p[1]['root'], cv_p[1]['bias'].reshape(1, 16),
        cv_d[1]['root'], cv_d[1]['bias'].reshape(1, 16))

    agg2p, agg2d = _sc_edge()(zs, t2p, t2d,
                              h2p.reshape(EP * 16), h2d.reshape(EP * 16),
                              srcp, dstp, srcd, dstd)

    return _final(agg2p[:, :16], agg2d[:, :16], r2p, r2d, segp, segd,
                  attp, attd, params)
